# Initial kernel scaffold; baseline (speedup 1.0000x reference)
import jax
import jax.numpy as jnp
from jax.experimental import pallas as pl


def kernel(inputs, embed_weight):
    raise NotImplementedError("write your pallas kernel here")


# speedup vs baseline: 1.0870x; 1.0870x over previous
import functools

import jax
import jax.numpy as jnp
from jax import lax
from jax.experimental import pallas as pl
from jax.experimental.pallas import tpu as pltpu
from jax.experimental.pallas import tpu_sc as plsc

_NC = 2
_NS = 16
_NW = _NC * _NS

_B = 16384
_N = _B + 1
_D = 64
_BPW = _B // _NW

_mesh = plsc.VectorSubcoreMesh(
    core_axis_name="c", subcore_axis_name="s", num_cores=_NC, num_subcores=_NS
)


@functools.partial(
    pl.kernel,
    mesh=_mesh,
    out_type=jax.ShapeDtypeStruct((2, _N, _D), jnp.float32),
    compiler_params=pltpu.CompilerParams(use_tc_tiling_on_sc=False),
    scratch_types=[
        pltpu.VMEM((_BPW,), jnp.int32),
        pltpu.VMEM((_BPW, _D), jnp.float32),
        pltpu.VMEM((8,), jnp.int32),
        pltpu.VMEM((8, _D), jnp.float32),
        pltpu.SemaphoreType.DMA,
    ],
)
def _embed_lookup(idx_hbm, table_hbm, out_hbm, idx_v, rows_v, idx8_v, rows8_v, sem):
    wid = lax.axis_index("s") * _NC + lax.axis_index("c")
    base = wid * _BPW
    pltpu.sync_copy(idx_hbm.at[pl.ds(base, _BPW)], idx_v)
    pltpu.async_copy(table_hbm.at[idx_v], rows_v, sem).wait()
    pltpu.sync_copy(rows_v, out_hbm.at[0, pl.ds(base, _BPW)])
    pltpu.sync_copy(rows_v, out_hbm.at[1, pl.ds(base, _BPW)])

    @pl.when(wid == 0)
    def _():
        pltpu.sync_copy(idx_hbm.at[pl.ds(_B, 8)], idx8_v)
        pltpu.async_copy(table_hbm.at[idx8_v], rows8_v, sem).wait()
        pltpu.sync_copy(rows8_v.at[pl.ds(0, 1)], out_hbm.at[0, pl.ds(_B, 1)])
        pltpu.sync_copy(rows8_v.at[pl.ds(0, 1)], out_hbm.at[1, pl.ds(_B, 1)])


def kernel(inputs, embed_weight):
    idx = inputs.reshape(-1).astype(jnp.int32)
    padded_idx = jnp.concatenate(
        [jnp.zeros((1,), jnp.int32), idx, jnp.zeros((7,), jnp.int32)]
    )
    out = _embed_lookup(padded_idx, embed_weight)
    return out.reshape(2, _N, 1, _D)
